# Initial kernel scaffold; baseline (speedup 1.0000x reference)
import jax
import jax.numpy as jnp
from jax.experimental import pallas as pl


def kernel(input_ids, word_table, pos_table):
    raise NotImplementedError("write your pallas kernel here")


# speedup vs baseline: 1.0277x; 1.0277x over previous
import functools

import jax
import jax.numpy as jnp
from jax import lax
from jax.experimental import pallas as pl
from jax.experimental.pallas import tpu as pltpu
from jax.experimental.pallas import tpu_sc as plsc

_NC, _NS = 2, 16
_NW = _NC * _NS
_G = 128


def kernel(input_ids, word_table, pos_table):
    B, S = input_ids.shape
    V, D = word_table.shape
    N = B * S
    rows_per_w = N // _NW
    n_iter = rows_per_w // _G

    ids_flat = input_ids.reshape(N)

    mesh = plsc.VectorSubcoreMesh(core_axis_name="c", subcore_axis_name="s")

    @functools.partial(
        pl.kernel,
        out_type=jax.ShapeDtypeStruct((N, D), jnp.float32),
        mesh=mesh,
        scratch_types=[
            pltpu.VMEM((_G,), jnp.int32),
            pltpu.VMEM((_G, D), jnp.float32),
            pltpu.SemaphoreType.DMA,
        ],
    )
    def emb(ids_hbm, wt_hbm, pt_hbm, out_hbm, idx_v, rows_v, sem):
        wid = lax.axis_index("s") * _NC + lax.axis_index("c")
        base = wid * rows_per_w
        pos_base = base % S

        def body(i, carry):
            cbase = base + i * _G
            pbase = pos_base + i * _G
            pltpu.sync_copy(ids_hbm.at[pl.ds(cbase, _G)], idx_v)
            pltpu.sync_copy(pt_hbm.at[pl.ds(pbase, _G)], rows_v)
            pltpu.async_copy(wt_hbm.at[idx_v], rows_v, sem, add=True).wait()
            pltpu.sync_copy(rows_v, out_hbm.at[pl.ds(cbase, _G)])
            return carry

        lax.fori_loop(0, n_iter, body, 0)

    out = emb(ids_flat, word_table, pos_table)
    return out.reshape(B, S, D)


# speedup vs baseline: 1.3333x; 1.2973x over previous
import functools

import jax
import jax.numpy as jnp
from jax import lax
from jax.experimental import pallas as pl
from jax.experimental.pallas import tpu as pltpu
from jax.experimental.pallas import tpu_sc as plsc

_NC, _NS = 2, 16
_NW = _NC * _NS
_G = 128
_CB = 2
_NBUF = 3


def kernel(input_ids, word_table, pos_table):
    B, S = input_ids.shape
    V, D = word_table.shape
    N = B * S
    NBLK = N // _G
    PBLK = S // _G
    blocks_per_w = NBLK // _NW
    niter = blocks_per_w // _CB

    ids_r = input_ids.reshape(NBLK, _G)
    pos_r = pos_table.reshape(PBLK, _G, D)

    mesh = plsc.VectorSubcoreMesh(core_axis_name="c", subcore_axis_name="s")

    scratch = (
        [pltpu.VMEM((_CB, _G), jnp.int32) for _ in range(_NBUF)]
        + [pltpu.VMEM((_CB, _G, D), jnp.float32) for _ in range(_NBUF)]
        + [pltpu.SemaphoreType.DMA for _ in range(3 * _NBUF)]
    )

    @functools.partial(
        pl.kernel,
        out_type=jax.ShapeDtypeStruct((NBLK, _G, D), jnp.float32),
        mesh=mesh,
        scratch_types=scratch,
    )
    def emb(ids_hbm, wt_hbm, pt_hbm, out_hbm, *sc):
        idx_bufs = sc[0:_NBUF]
        row_bufs = sc[_NBUF:2 * _NBUF]
        lsems = sc[2 * _NBUF:3 * _NBUF]
        gsems = sc[3 * _NBUF:4 * _NBUF]
        osems = sc[4 * _NBUF:5 * _NBUF]

        wid = lax.axis_index("s") * _NC + lax.axis_index("c")
        blk0 = wid * blocks_per_w
        pblk0 = blk0 % PBLK

        def start_loads(i):
            b = i % _NBUF
            blk = blk0 + i * _CB
            pblk = pblk0 + i * _CB
            return (
                pltpu.async_copy(ids_hbm.at[pl.ds(blk, _CB)], idx_bufs[b],
                                 lsems[b]),
                pltpu.async_copy(pt_hbm.at[pl.ds(pblk, _CB)], row_bufs[b],
                                 lsems[b]),
            )

        loads, stores = {}, {}
        for i in range(min(_NBUF, niter)):
            loads[i] = start_loads(i)

        for i in range(niter):
            b = i % _NBUF
            for d in loads[i]:
                d.wait()
            gds = [
                pltpu.async_copy(wt_hbm.at[idx_bufs[b].at[k]],
                                 row_bufs[b].at[k], gsems[b], add=True)
                for k in range(_CB)
            ]
            j = i + 1
            if _NBUF <= j < niter:
                stores[j - _NBUF].wait()
                loads[j] = start_loads(j)
            for d in gds:
                d.wait()
            stores[i] = pltpu.async_copy(
                row_bufs[b], out_hbm.at[pl.ds(blk0 + i * _CB, _CB)], osems[b])

        for i in range(max(0, niter - _NBUF), niter):
            stores[i].wait()

    out = emb(ids_r, word_table, pos_r)
    return out.reshape(B, S, D)


# speedup vs baseline: 1.3478x; 1.0109x over previous
import functools

import jax
import jax.numpy as jnp
from jax import lax
from jax.experimental import pallas as pl
from jax.experimental.pallas import tpu as pltpu
from jax.experimental.pallas import tpu_sc as plsc

_NC, _NS = 2, 16
_NW = _NC * _NS
_G = 128
_CB = 2
_NBUF = 3


def kernel(input_ids, word_table, pos_table):
    B, S = input_ids.shape
    V, D = word_table.shape
    N = B * S
    NBLK = N // _G
    PBLK = S // _G
    blocks_per_w = NBLK // _NW
    niter = blocks_per_w // _CB

    pos_r = pos_table.reshape(PBLK, _G, D)

    mesh = plsc.VectorSubcoreMesh(core_axis_name="c", subcore_axis_name="s")

    scratch = (
        [pltpu.VMEM((_CB * _G,), jnp.int32) for _ in range(_NBUF)]
        + [pltpu.VMEM((_CB, _G, D), jnp.float32) for _ in range(_NBUF)]
        + [pltpu.SemaphoreType.DMA for _ in range(3 * _NBUF)]
    )

    @functools.partial(
        pl.kernel,
        out_type=jax.ShapeDtypeStruct((NBLK, _G, D), jnp.float32),
        mesh=mesh,
        scratch_types=scratch,
    )
    def emb(ids_hbm, wt_hbm, pt_hbm, out_hbm, *sc):
        idx_bufs = sc[0:_NBUF]
        row_bufs = sc[_NBUF:2 * _NBUF]
        lsems = sc[2 * _NBUF:3 * _NBUF]
        gsems = sc[3 * _NBUF:4 * _NBUF]
        osems = sc[4 * _NBUF:5 * _NBUF]

        wid = lax.axis_index("s") * _NC + lax.axis_index("c")
        blk0 = wid * blocks_per_w
        pblk0 = blk0 % PBLK
        bat = blk0 // PBLK
        soff = (blk0 % PBLK) * _G

        def start_loads(i):
            b = i % _NBUF
            return (
                pltpu.async_copy(
                    ids_hbm.at[bat, pl.ds(soff + i * _CB * _G, _CB * _G)],
                    idx_bufs[b], lsems[b]),
                pltpu.async_copy(pt_hbm.at[pl.ds(pblk0 + i * _CB, _CB)],
                                 row_bufs[b], lsems[b]),
            )

        def start_gathers(i):
            b = i % _NBUF
            return [
                pltpu.async_copy(wt_hbm.at[idx_bufs[b].at[pl.ds(k * _G, _G)]],
                                 row_bufs[b].at[k], gsems[b], add=True)
                for k in range(_CB)
            ]

        loads, gath, stores = {}, {}, {}
        for i in range(min(_NBUF, niter)):
            loads[i] = start_loads(i)
        for d in loads[0]:
            d.wait()
        gath[0] = start_gathers(0)

        for i in range(niter):
            b = i % _NBUF
            if i + 1 < niter:
                for d in loads[i + 1]:
                    d.wait()
                gath[i + 1] = start_gathers(i + 1)
            for d in gath[i]:
                d.wait()
            stores[i] = pltpu.async_copy(
                row_bufs[b], out_hbm.at[pl.ds(blk0 + i * _CB, _CB)], osems[b])
            j = i + 2
            if _NBUF <= j < niter:
                stores[j - _NBUF].wait()
                loads[j] = start_loads(j)

        for i in range(max(0, niter - _NBUF), niter):
            stores[i].wait()

    out = emb(input_ids, word_table, pos_r)
    return out.reshape(B, S, D)


# speedup vs baseline: 1.4148x; 1.0497x over previous
import functools

import jax
import jax.numpy as jnp
from jax import lax
from jax.experimental import pallas as pl
from jax.experimental.pallas import tpu as pltpu
from jax.experimental.pallas import tpu_sc as plsc

_NC, _NS = 2, 16
_NW = _NC * _NS
_G = 128
_CB = 2
_L = 16


def kernel(input_ids, word_table, pos_table):
    B, S = input_ids.shape
    V, D = word_table.shape
    N = B * S
    NBLK = N // _G
    PBLK = S // _G
    R = _CB * _G
    assert _NW * R == S
    niter = B

    pos_r = pos_table.reshape(PBLK, _G, D)

    mesh = plsc.VectorSubcoreMesh(core_axis_name="c", subcore_axis_name="s")

    scratch = (
        [pltpu.VMEM((R,), jnp.int32) for _ in range(niter)]
        + [pltpu.VMEM((_CB, _G, D), jnp.float32) for _ in range(2)]
        + [pltpu.VMEM((_CB, _G, D), jnp.float32)]
        + [pltpu.SemaphoreType.DMA for _ in range(niter + 2 + 2 + 1)]
    )

    @functools.partial(
        pl.kernel,
        out_type=jax.ShapeDtypeStruct((NBLK, _G, D), jnp.float32),
        mesh=mesh,
        scratch_types=scratch,
    )
    def emb(ids_hbm, wt_hbm, pt_hbm, out_hbm, *sc):
        idx_bufs = sc[0:niter]
        row_bufs = sc[niter:niter + 2]
        pos_buf = sc[niter + 2]
        lsems = sc[niter + 3:2 * niter + 3]
        gsems = sc[2 * niter + 3:2 * niter + 5]
        osems = sc[2 * niter + 5:2 * niter + 7]
        psem = sc[2 * niter + 7]

        wid = lax.axis_index("s") * _NC + lax.axis_index("c")
        pblk0 = wid * _CB

        idxd = [
            pltpu.async_copy(ids_hbm.at[c, pl.ds(pblk0 * _G, R)],
                             idx_bufs[c], lsems[c])
            for c in range(niter)
        ]
        posd = pltpu.async_copy(pt_hbm.at[pl.ds(pblk0, _CB)], pos_buf, psem)

        def start_gathers(i):
            b = i % 2
            return [
                pltpu.async_copy(wt_hbm.at[idx_bufs[i].at[pl.ds(k * _G, _G)]],
                                 row_bufs[b].at[k], gsems[b])
                for k in range(_CB)
            ]

        def add_pos(i):
            b = i % 2

            def body(r, carry):
                for blkk in range(_CB):
                    for c0 in range(0, D, _L):
                        v = pos_buf[blkk, r, pl.ds(c0, _L)]
                        plsc.addupdate(
                            row_bufs[b].at[blkk, r, pl.ds(c0, _L)], v)
                return carry

            lax.fori_loop(0, _G, body, 0)

        gath, stores = {}, {}
        idxd[0].wait()
        gath[0] = start_gathers(0)

        for i in range(niter):
            b = i % 2
            if i + 1 < niter:
                idxd[i + 1].wait()
                if i >= 1:
                    stores[i - 1].wait()
                gath[i + 1] = start_gathers(i + 1)
            for d in gath[i]:
                d.wait()
            if i == 0:
                posd.wait()
            add_pos(i)
            stores[i] = pltpu.async_copy(
                row_bufs[b], out_hbm.at[pl.ds(i * PBLK + pblk0, _CB)],
                osems[b])

        for i in range(max(0, niter - 2), niter):
            stores[i].wait()

    out = emb(input_ids, word_table, pos_r)
    return out.reshape(B, S, D)


# speedup vs baseline: 1.4335x; 1.0132x over previous
import functools

import jax
import jax.numpy as jnp
from jax import lax
from jax.experimental import pallas as pl
from jax.experimental.pallas import tpu as pltpu
from jax.experimental.pallas import tpu_sc as plsc

_NC, _NS = 2, 16
_NW = _NC * _NS
_G = 128
_CB = 2
_L = 16


def kernel(input_ids, word_table, pos_table):
    B, S = input_ids.shape
    V, D = word_table.shape
    N = B * S
    NBLK = N // _G
    PBLK = S // _G
    R = _CB * _G
    assert _NW * R == S
    niter = B

    pos_r = pos_table.reshape(PBLK, _G, D)

    mesh = plsc.VectorSubcoreMesh(core_axis_name="c", subcore_axis_name="s")

    scratch = (
        [pltpu.VMEM((R,), jnp.int32) for _ in range(niter)]
        + [pltpu.VMEM((_CB, _G, D), jnp.float32) for _ in range(2)]
        + [pltpu.VMEM((_CB, _G, D), jnp.float32)]
        + [pltpu.SemaphoreType.DMA for _ in range(niter + 2 + 2 + 1)]
    )

    @functools.partial(
        pl.kernel,
        out_type=jax.ShapeDtypeStruct((NBLK, _G, D), jnp.float32),
        mesh=mesh,
        scratch_types=scratch,
    )
    def emb(ids_hbm, wt_hbm, pt_hbm, out_hbm, *sc):
        idx_bufs = sc[0:niter]
        row_bufs = sc[niter:niter + 2]
        pos_buf = sc[niter + 2]
        lsems = sc[niter + 3:2 * niter + 3]
        gsems = sc[2 * niter + 3:2 * niter + 5]
        osems = sc[2 * niter + 5:2 * niter + 7]
        psem = sc[2 * niter + 7]

        wid = lax.axis_index("s") * _NC + lax.axis_index("c")
        pblk0 = wid * _CB

        idxd = [
            pltpu.async_copy(ids_hbm.at[c, pl.ds(pblk0 * _G, R)],
                             idx_bufs[c], lsems[c])
            for c in range(niter)
        ]
        posd = pltpu.async_copy(pt_hbm.at[pl.ds(pblk0, _CB)], pos_buf, psem)

        def start_gathers(i):
            b = i % 2
            return [
                pltpu.async_copy(wt_hbm.at[idx_bufs[i].at[pl.ds(k * _G, _G)]],
                                 row_bufs[b].at[k], gsems[b])
                for k in range(_CB)
            ]

        def add_pos_block(i, blkk):
            b = i % 2

            @plsc.parallel_loop(0, _G, unroll=4)
            def body(r):
                for c0 in range(0, D, _L):
                    v = pos_buf[blkk, r, pl.ds(c0, _L)]
                    plsc.addupdate(
                        row_bufs[b].at[blkk, r, pl.ds(c0, _L)], v)

        gath, stores = {}, {}
        idxd[0].wait()
        gath[0] = start_gathers(0)

        for i in range(niter):
            b = i % 2
            if i + 1 < niter:
                idxd[i + 1].wait()
                if i >= 1:
                    for d in stores[i - 1]:
                        d.wait()
                gath[i + 1] = start_gathers(i + 1)
            if i == 0:
                posd.wait()
            blk_stores = []
            for k in range(_CB):
                gath[i][k].wait()
                add_pos_block(i, k)
                blk_stores.append(pltpu.async_copy(
                    row_bufs[b].at[k],
                    out_hbm.at[i * PBLK + pblk0 + k], osems[b]))
            stores[i] = blk_stores

        for i in range(max(0, niter - 2), niter):
            for d in stores[i]:
                d.wait()

    out = emb(input_ids, word_table, pos_r)
    return out.reshape(B, S, D)


# speedup vs baseline: 1.4403x; 1.0047x over previous
import functools

import jax
import jax.numpy as jnp
from jax import lax
from jax.experimental import pallas as pl
from jax.experimental.pallas import tpu as pltpu
from jax.experimental.pallas import tpu_sc as plsc

_NC, _NS = 2, 16
_NW = _NC * _NS
_G = 128
_CB = 2
_L = 16


def kernel(input_ids, word_table, pos_table):
    B, S = input_ids.shape
    V, D = word_table.shape
    N = B * S
    NBLK = N // _G
    PBLK = S // _G
    R = _CB * _G
    assert _NW * R == S
    niter = B

    pos_r = pos_table.reshape(PBLK, _G, D)

    mesh = plsc.VectorSubcoreMesh(core_axis_name="c", subcore_axis_name="s")

    scratch = (
        [pltpu.VMEM((R,), jnp.int32) for _ in range(niter)]
        + [pltpu.VMEM((_CB, _G, D), jnp.float32) for _ in range(2)]
        + [pltpu.VMEM((_CB, _G, D), jnp.float32)]
        + [pltpu.SemaphoreType.DMA for _ in range(niter + 2 + 2 + 1)]
    )

    @functools.partial(
        pl.kernel,
        out_type=jax.ShapeDtypeStruct((B, S, D), jnp.float32),
        mesh=mesh,
        scratch_types=scratch,
    )
    def emb(ids_hbm, wt_hbm, pt_hbm, out_hbm, *sc):
        idx_bufs = sc[0:niter]
        row_bufs = sc[niter:niter + 2]
        pos_buf = sc[niter + 2]
        lsems = sc[niter + 3:2 * niter + 3]
        gsems = sc[2 * niter + 3:2 * niter + 5]
        osems = sc[2 * niter + 5:2 * niter + 7]
        psem = sc[2 * niter + 7]

        wid = lax.axis_index("s") * _NC + lax.axis_index("c")
        pblk0 = wid * _CB

        idxd = [
            pltpu.async_copy(ids_hbm.at[c, pl.ds(pblk0 * _G, R)],
                             idx_bufs[c], lsems[c])
            for c in range(niter)
        ]
        posd = pltpu.async_copy(pt_hbm.at[pl.ds(pblk0, _CB)], pos_buf, psem)

        def start_gathers(i):
            b = i % 2
            return [
                pltpu.async_copy(wt_hbm.at[idx_bufs[i].at[pl.ds(k * _G, _G)]],
                                 row_bufs[b].at[k], gsems[b])
                for k in range(_CB)
            ]

        def add_pos_block(i, blkk):
            b = i % 2

            @plsc.parallel_loop(0, _G, unroll=4)
            def body(r):
                for c0 in range(0, D, _L):
                    v = pos_buf[blkk, r, pl.ds(c0, _L)]
                    plsc.addupdate(
                        row_bufs[b].at[blkk, r, pl.ds(c0, _L)], v)

        gath, stores = {}, {}
        idxd[0].wait()
        gath[0] = start_gathers(0)

        for i in range(niter):
            b = i % 2
            if i + 1 < niter:
                idxd[i + 1].wait()
                if i >= 1:
                    for d in stores[i - 1]:
                        d.wait()
                gath[i + 1] = start_gathers(i + 1)
            if i == 0:
                posd.wait()
            blk_stores = []
            for k in range(_CB):
                gath[i][k].wait()
                add_pos_block(i, k)
                blk_stores.append(pltpu.async_copy(
                    row_bufs[b].at[k],
                    out_hbm.at[i, pl.ds((pblk0 + k) * _G, _G)], osems[b]))
            stores[i] = blk_stores

        for i in range(max(0, niter - 2), niter):
            for d in stores[i]:
                d.wait()

    return emb(input_ids, word_table, pos_r)
